# baseline (speedup 1.0000x reference)
import functools

import jax
import jax.numpy as jnp
from jax import lax
from jax.experimental import pallas as pl
from jax.experimental.pallas import tpu as pltpu
from jax.experimental.pallas import tpu_sc as plsc

N = 10000
E = 320000
D_IN = 128
F = 32
OUT = 128

NC = 2
NS = 16
NW = NC * NS
CHUNK = 128
N_PAD = 10112
ROWS_PER_TILE = N_PAD // NS

FG = 8
NGRP = F // FG
NR = NW // NGRP
SUP = 16
EDGES_PER_R = 40960
NCHUNK_R = EDGES_PER_R // CHUNK
NSUP = NCHUNK_R // SUP
E_PAD = EDGES_PER_R * NR

EDGES_PER_W = E_PAD // NW
NCHUNK_W = EDGES_PER_W // CHUNK


def _deg_body(dst_hbm, zeros_hbm, ones_hbm, out_hbm, dst_v, ones_v, deg_sh):
    c = lax.axis_index("c")
    s = lax.axis_index("s")
    wid = s * NC + c
    row0 = s * ROWS_PER_TILE
    pltpu.sync_copy(zeros_hbm.at[pl.ds(row0, ROWS_PER_TILE)],
                    deg_sh.at[pl.ds(row0, ROWS_PER_TILE)])
    pltpu.sync_copy(dst_hbm.at[wid], dst_v)
    pltpu.sync_copy(ones_hbm, ones_v)
    plsc.subcore_barrier()

    def body(j, _):
        pltpu.sync_copy(ones_v, deg_sh.at[dst_v.at[j]], add=True)
        return ()

    lax.fori_loop(0, NCHUNK_W, body, ())
    plsc.subcore_barrier()
    pltpu.sync_copy(deg_sh.at[pl.ds(row0, ROWS_PER_TILE)],
                    out_hbm.at[c, pl.ds(row0, ROWS_PER_TILE)])


NSLOT = 4
NPF = N_PAD * FG
OCT = N_PAD // 8
OPF = OCT * FG


def _agg_body(src_hbm, dst8_hbm, hp_hbm, zeros_hbm, part_hbm, out_hbm,
              srcv, dstv, rows0, rows1, rows2, rows3, acc, acc2,
              semi0, semi1, semg0, semg1, semg2, semg3):
    c = lax.axis_index("c")
    s = lax.axis_index("s")
    g = 2 * c + s // 8
    r = lax.rem(s, 8)
    semi = [semi0, semi1]
    semg = [semg0, semg1, semg2, semg3]
    rows = [rows0, rows1, rows2, rows3]

    pltpu.sync_copy(zeros_hbm, acc)

    iota = lax.iota(jnp.int32, 16)
    feat = jnp.bitwise_and(iota, 7)
    half = lax.shift_right_logical(iota, 3)
    pats = [2 * j + half for j in range(8)]
    evecs = [[t * 16 + 2 * j + half for j in range(8)] for t in range(8)]

    def fire(b, ci, k):
        pltpu.async_copy(hp_hbm.at[g].at[srcv.at[b, ci]], rows[k], semg[k])

    def wait(b, ci, k):
        pltpu.make_async_copy(
            hp_hbm.at[g].at[srcv.at[b, ci]], rows[k], semg[k]).wait()

    pltpu.async_copy(src_hbm.at[r, 0], srcv.at[0], semi0)
    pltpu.async_copy(dst8_hbm.at[r, 0], dstv.at[0], semi0)
    pltpu.make_async_copy(src_hbm.at[r, 0], srcv.at[0], semi0).wait()
    pltpu.make_async_copy(dst8_hbm.at[r, 0], dstv.at[0], semi0).wait()
    pltpu.async_copy(src_hbm.at[r, 1], srcv.at[1], semi1)
    pltpu.async_copy(dst8_hbm.at[r, 1], dstv.at[1], semi1)
    for k in range(NSLOT - 1):
        fire(0, k, k)

    def scatter_chunk(b, ci, k):
        for t in range(8):
            d16 = dstv[b, pl.ds(ci * CHUNK + t * 16, 16)]
            addrs = [jnp.bitwise_or(
                d16.at[pats[j]].get(mode="promise_in_bounds"), feat)
                for j in range(8)]
            vs = [plsc.load_gather(rows[k], [evecs[t][j], feat])
                  for j in range(8)]
            for j in range(8):
                plsc.addupdate_scatter(acc, [addrs[j]], vs[j])

    def super_body(sj, _):
        for b in range(2):
            sb = sj * 2 + b

            def chunk_body(cj, _):
                for k in range(NSLOT):
                    ci = cj * NSLOT + k
                    wait(b, ci, k)

                    @pl.when(ci + NSLOT - 1 < SUP)
                    def _():
                        fire(b, ci + NSLOT - 1, (k + NSLOT - 1) % NSLOT)

                    scatter_chunk(b, ci, k)
                return ()

            lax.fori_loop(0, SUP // NSLOT, chunk_body, ())

            @pl.when(sb + 1 < NSUP)
            def _():
                nb = 1 - b
                pltpu.make_async_copy(
                    src_hbm.at[r, sb + 1], srcv.at[nb], semi[nb]).wait()
                pltpu.make_async_copy(
                    dst8_hbm.at[r, sb + 1], dstv.at[nb], semi[nb]).wait()

                @pl.when(sb + 2 < NSUP)
                def _():
                    pltpu.async_copy(src_hbm.at[r, sb + 2], srcv.at[b],
                                     semi[b])
                    pltpu.async_copy(dst8_hbm.at[r, sb + 2], dstv.at[b],
                                     semi[b])

                for k in range(NSLOT - 1):
                    fire(nb, k, k)
        return ()

    lax.fori_loop(0, NSUP // 2, super_body, ())

    pltpu.sync_copy(acc, part_hbm.at[r, g])
    plsc.subcore_barrier()

    oct_ = lax.rem(s, 8)
    gm = 2 * c + s // 8
    for rp in range(NR):
        pltpu.sync_copy(part_hbm.at[rp, gm, pl.ds(oct_ * OPF, OPF)],
                        acc.at[pl.ds(rp * OPF, OPF)])

    def red_body(pos, _):
        v = acc[pl.ds(pos * 16, 16)]
        for rp in range(1, NR):
            v = v + acc[pl.ds(rp * OPF + pos * 16, 16)]
        rowv = pos * 2 + half
        plsc.store_scatter(acc2, [rowv, feat], v)
        return ()

    lax.fori_loop(0, OPF // 16, red_body, ())
    pltpu.sync_copy(acc2, out_hbm.at[pl.ds(oct_ * OCT, OCT),
                                     pl.ds(gm * FG, FG)])


@functools.lru_cache(maxsize=None)
def _sc_kernels():
    mesh = plsc.VectorSubcoreMesh(
        core_axis_name="c", subcore_axis_name="s",
        num_cores=NC, num_subcores=NS)
    deg_k = pl.kernel(
        _deg_body,
        out_type=jax.ShapeDtypeStruct((NC, N_PAD, 8), jnp.float32),
        mesh=mesh,
        compiler_params=pltpu.CompilerParams(use_tc_tiling_on_sc=False),
        scratch_types=[
            pltpu.VMEM((NCHUNK_W, CHUNK), jnp.int32),
            pltpu.VMEM((CHUNK, 8), jnp.float32),
            pltpu.VMEM_SHARED((N_PAD, 8), jnp.float32),
        ],
    )
    agg_k = pl.kernel(
        _agg_body,
        out_type=(
            jax.ShapeDtypeStruct((NR, NGRP, NPF), jnp.float32),
            jax.ShapeDtypeStruct((N_PAD, F), jnp.float32),
        ),
        mesh=mesh,
        compiler_params=pltpu.CompilerParams(
            use_tc_tiling_on_sc=False, needs_layout_passes=False),
        scratch_types=[
            pltpu.VMEM((2, SUP, CHUNK), jnp.int32),
            pltpu.VMEM((2, SUP * CHUNK), jnp.int32),
            pltpu.VMEM((CHUNK, FG), jnp.float32),
            pltpu.VMEM((CHUNK, FG), jnp.float32),
            pltpu.VMEM((CHUNK, FG), jnp.float32),
            pltpu.VMEM((CHUNK, FG), jnp.float32),
            pltpu.VMEM((NPF,), jnp.float32),
            pltpu.VMEM((OCT, FG), jnp.float32),
            pltpu.SemaphoreType.DMA,
            pltpu.SemaphoreType.DMA,
            pltpu.SemaphoreType.DMA,
            pltpu.SemaphoreType.DMA,
            pltpu.SemaphoreType.DMA,
            pltpu.SemaphoreType.DMA,
        ],
    )
    return deg_k, agg_k


_BLK = 1264


def _tcmm_body(x_ref, w_ref, h_ref):
    h_ref[...] = jnp.dot(x_ref[...], w_ref[...],
                         preferred_element_type=jnp.float32)


def _tc_matmul1(x_pad, W1):
    return pl.pallas_call(
        _tcmm_body,
        grid=(N_PAD // _BLK,),
        in_specs=[
            pl.BlockSpec((_BLK, D_IN), lambda i: (i, 0)),
            pl.BlockSpec((D_IN, F), lambda i: (0, 0)),
        ],
        out_specs=pl.BlockSpec((_BLK, F), lambda i: (i, 0)),
        out_shape=jax.ShapeDtypeStruct((N_PAD, F), jnp.float32),
    )(x_pad, W1)


def _tc1_body(h_ref, degp_ref, hp_ref, dinv_ref):
    deg = degp_ref[0] + degp_ref[1] + 1.0
    dinv = lax.rsqrt(deg)
    hp_ref[...] = h_ref[...] * dinv
    dinv_ref[...] = dinv


def _tc_stage1(h1, degp):
    return pl.pallas_call(
        _tc1_body,
        grid=(N_PAD // _BLK,),
        in_specs=[
            pl.BlockSpec((_BLK, F), lambda i: (i, 0)),
            pl.BlockSpec((NC, _BLK, 1), lambda i: (0, i, 0)),
        ],
        out_specs=[
            pl.BlockSpec((_BLK, F), lambda i: (i, 0)),
            pl.BlockSpec((_BLK, 1), lambda i: (i, 0)),
        ],
        out_shape=[
            jax.ShapeDtypeStruct((N_PAD, F), jnp.float32),
            jax.ShapeDtypeStruct((N_PAD, 1), jnp.float32),
        ],
    )(h1, degp)


def _tc_combine_body(p_ref, hp_ref, dinv_ref, b_ref, g_ref, be_ref, w_ref,
                     out_ref):
    dinv = dinv_ref[...]
    agg = (p_ref[...] + hp_ref[...]) * dinv + b_ref[...]
    h = jnp.maximum(agg, 0.0)
    mu = jnp.mean(h, axis=1, keepdims=True)
    var = jnp.mean((h - mu) * (h - mu), axis=1, keepdims=True)
    hn = (h - mu) * lax.rsqrt(var + 1e-5) * g_ref[...] + be_ref[...]
    out_ref[...] = jnp.dot(
        hn, w_ref[...], preferred_element_type=jnp.float32) * dinv


def _tc_combine(p, hp, dinv2, b, g, be, Wn):
    return pl.pallas_call(
        _tc_combine_body,
        grid=(N_PAD // _BLK,),
        in_specs=[
            pl.BlockSpec((_BLK, F), lambda i: (i, 0)),
            pl.BlockSpec((_BLK, F), lambda i: (i, 0)),
            pl.BlockSpec((_BLK, 1), lambda i: (i, 0)),
            pl.BlockSpec((1, F), lambda i: (0, 0)),
            pl.BlockSpec((1, F), lambda i: (0, 0)),
            pl.BlockSpec((1, F), lambda i: (0, 0)),
            pl.BlockSpec((F, F), lambda i: (0, 0)),
        ],
        out_specs=pl.BlockSpec((_BLK, F), lambda i: (i, 0)),
        out_shape=jax.ShapeDtypeStruct((N_PAD, F), jnp.float32),
    )(p, hp, dinv2, b, g, be, Wn)


def _tc_final_body(p_ref, hp_ref, dinv_ref, b3_ref, wp1_ref, bp1_ref,
                   wp2_ref, bp2_ref, emb_ref, lsm_ref):
    dinv = dinv_ref[...]
    emb = (p_ref[...] + hp_ref[...]) * dinv + b3_ref[...]
    emb_ref[...] = emb
    r = jnp.maximum(emb, 0.0)
    t = jnp.dot(r, wp1_ref[...], preferred_element_type=jnp.float32)
    t = t + bp1_ref[...]
    u = jnp.dot(t, wp2_ref[...], preferred_element_type=jnp.float32)
    u = u + bp2_ref[...]
    m = jnp.max(u, axis=1, keepdims=True)
    lse = jnp.log(jnp.sum(jnp.exp(u - m), axis=1, keepdims=True)) + m
    lsm_ref[...] = u - lse


_FBLK = 1000


def _tc_final(p, hp, dinv2, b3, Wp1, bp1, Wp2, bp2):
    return pl.pallas_call(
        _tc_final_body,
        grid=(N // _FBLK,),
        in_specs=[
            pl.BlockSpec((_FBLK, F), lambda i: (i, 0)),
            pl.BlockSpec((_FBLK, F), lambda i: (i, 0)),
            pl.BlockSpec((_FBLK, 1), lambda i: (i, 0)),
            pl.BlockSpec((1, F), lambda i: (0, 0)),
            pl.BlockSpec((F, F), lambda i: (0, 0)),
            pl.BlockSpec((1, F), lambda i: (0, 0)),
            pl.BlockSpec((F, OUT), lambda i: (0, 0)),
            pl.BlockSpec((1, OUT), lambda i: (0, 0)),
        ],
        out_specs=[
            pl.BlockSpec((_FBLK, F), lambda i: (i, 0)),
            pl.BlockSpec((_FBLK, OUT), lambda i: (i, 0)),
        ],
        out_shape=[
            jax.ShapeDtypeStruct((N, F), jnp.float32),
            jax.ShapeDtypeStruct((N, OUT), jnp.float32),
        ],
    )(p, hp, dinv2, b3, Wp1, bp1, Wp2, bp2)


def _group(hp):
    return jnp.transpose(hp.reshape(N_PAD, NGRP, FG), (1, 0, 2))


def kernel(x, edge_index, W1, b1, g1, be1, W2, b2, g2, be2, W3, b3,
           Wp1, bp1, Wp2, bp2):
    src = edge_index[0]
    dst = edge_index[1]
    pad_r = EDGES_PER_R - E // NR
    src_r = jnp.pad(src.reshape(NR, E // NR), ((0, 0), (0, pad_r)))
    src_r = src_r.reshape(NR, NSUP, SUP, CHUNK)
    dst8_r = jnp.pad((dst * 8).reshape(NR, E // NR), ((0, 0), (0, pad_r)),
                     constant_values=N * 8)
    dst8_r = dst8_r.reshape(NR, NSUP, SUP * CHUNK)

    dst_w = jnp.pad(dst, (0, E_PAD - E),
                    constant_values=N).reshape(NW, NCHUNK_W, CHUNK)

    zeros8 = jnp.zeros((N_PAD, 8), jnp.float32)
    ones8 = jnp.ones((CHUNK, 8), jnp.float32)
    zerosG = jnp.zeros((NPF,), jnp.float32)
    x_pad = jnp.pad(x, ((0, N_PAD - N), (0, 0)))

    _deg_kernel, _agg_kernel = _sc_kernels()
    h1 = _tc_matmul1(x_pad, W1)
    degp8 = _deg_kernel(dst_w, zeros8, ones8)
    degp = degp8[:, :, :1]

    hp1, dinv2 = _tc_stage1(h1, degp)

    p1 = _agg_kernel(src_r, dst8_r, _group(hp1), zerosG)[1]
    hp2 = _tc_combine(p1, hp1, dinv2, b1.reshape(1, F), g1.reshape(1, F),
                      be1.reshape(1, F), W2)

    p2 = _agg_kernel(src_r, dst8_r, _group(hp2), zerosG)[1]
    hp3 = _tc_combine(p2, hp2, dinv2, b2.reshape(1, F), g2.reshape(1, F),
                      be2.reshape(1, F), W3)

    p3 = _agg_kernel(src_r, dst8_r, _group(hp3), zerosG)[1]
    emb, lsm = _tc_final(p3, hp3, dinv2, b3.reshape(1, F), Wp1,
                         bp1.reshape(1, F), Wp2, bp2.reshape(1, OUT))
    return (emb, lsm)


# speedup vs baseline: 1.0325x; 1.0325x over previous
import functools

import jax
import jax.numpy as jnp
from jax import lax
from jax.experimental import pallas as pl
from jax.experimental.pallas import tpu as pltpu
from jax.experimental.pallas import tpu_sc as plsc

N = 10000
E = 320000
D_IN = 128
F = 32
OUT = 128

NC = 2
NS = 16
NW = NC * NS
CHUNK = 128
N_PAD = 10112
ROWS_PER_TILE = N_PAD // NS

FG = 8
NGRP = F // FG
NR = NW // NGRP
SUP = 16
EDGES_PER_R = 40960
NCHUNK_R = EDGES_PER_R // CHUNK
NSUP = NCHUNK_R // SUP
E_PAD = EDGES_PER_R * NR

EDGES_PER_W = E_PAD // NW
NCHUNK_W = EDGES_PER_W // CHUNK


def _deg_body(dst_hbm, zeros_hbm, ones_hbm, out_hbm, dst_v, ones_v, deg_sh):
    c = lax.axis_index("c")
    s = lax.axis_index("s")
    wid = s * NC + c
    row0 = s * ROWS_PER_TILE
    pltpu.sync_copy(zeros_hbm.at[pl.ds(row0, ROWS_PER_TILE)],
                    deg_sh.at[pl.ds(row0, ROWS_PER_TILE)])
    pltpu.sync_copy(dst_hbm.at[wid], dst_v)
    pltpu.sync_copy(ones_hbm, ones_v)
    plsc.subcore_barrier()

    def body(j, _):
        pltpu.sync_copy(ones_v, deg_sh.at[dst_v.at[j]], add=True)
        return ()

    lax.fori_loop(0, NCHUNK_W, body, ())
    plsc.subcore_barrier()
    pltpu.sync_copy(deg_sh.at[pl.ds(row0, ROWS_PER_TILE)],
                    out_hbm.at[c, pl.ds(row0, ROWS_PER_TILE)])


NSLOT = 4
NPF = N_PAD * FG
OCT = N_PAD // 8
OPF = OCT * FG


def _agg_body(src_hbm, dst8_hbm, hp_hbm, zeros_hbm, part_hbm, out_hbm,
              srcv, dstv, rows0, rows1, rows2, rows3, acc, acc2,
              semi0, semi1, semg0, semg1, semg2, semg3):
    c = lax.axis_index("c")
    s = lax.axis_index("s")
    g = 2 * c + s // 8
    r = lax.rem(s, 8)
    semi = [semi0, semi1]
    semg = [semg0, semg1, semg2, semg3]
    rows = [rows0, rows1, rows2, rows3]

    pltpu.sync_copy(zeros_hbm, acc)

    iota = lax.iota(jnp.int32, 16)
    feat = jnp.bitwise_and(iota, 7)
    half = lax.shift_right_logical(iota, 3)
    pats = [2 * j + half for j in range(8)]
    evecs = [[t * 16 + 2 * j + half for j in range(8)] for t in range(8)]

    def fire(b, ci, k):
        pltpu.async_copy(hp_hbm.at[g].at[srcv.at[b, ci]], rows[k], semg[k])

    def wait(b, ci, k):
        pltpu.make_async_copy(
            hp_hbm.at[g].at[srcv.at[b, ci]], rows[k], semg[k]).wait()

    pltpu.async_copy(src_hbm.at[r, 0], srcv.at[0], semi0)
    pltpu.async_copy(dst8_hbm.at[r, 0], dstv.at[0], semi0)
    pltpu.make_async_copy(src_hbm.at[r, 0], srcv.at[0], semi0).wait()
    pltpu.make_async_copy(dst8_hbm.at[r, 0], dstv.at[0], semi0).wait()
    pltpu.async_copy(src_hbm.at[r, 1], srcv.at[1], semi1)
    pltpu.async_copy(dst8_hbm.at[r, 1], dstv.at[1], semi1)
    for k in range(NSLOT - 1):
        fire(0, k, k)

    def scatter_chunk(b, ci, k):
        for t in range(8):
            d16 = dstv[b, pl.ds(ci * CHUNK + t * 16, 16)]
            addrs = [jnp.bitwise_or(
                d16.at[pats[j]].get(mode="promise_in_bounds"), feat)
                for j in range(8)]
            vs = [plsc.load_gather(rows[k], [evecs[t][j], feat])
                  for j in range(8)]
            for j in range(8):
                plsc.addupdate_scatter(acc, [addrs[j]], vs[j])

    def super_body(sj, _):
        for b in range(2):
            sb = sj * 2 + b

            def chunk_body(cj, _):
                for k in range(NSLOT):
                    ci = cj * NSLOT + k
                    wait(b, ci, k)

                    @pl.when(ci + NSLOT - 1 < SUP)
                    def _():
                        fire(b, ci + NSLOT - 1, (k + NSLOT - 1) % NSLOT)

                    scatter_chunk(b, ci, k)
                return ()

            lax.fori_loop(0, SUP // NSLOT, chunk_body, ())

            @pl.when(sb + 1 < NSUP)
            def _():
                nb = 1 - b
                pltpu.make_async_copy(
                    src_hbm.at[r, sb + 1], srcv.at[nb], semi[nb]).wait()
                pltpu.make_async_copy(
                    dst8_hbm.at[r, sb + 1], dstv.at[nb], semi[nb]).wait()

                @pl.when(sb + 2 < NSUP)
                def _():
                    pltpu.async_copy(src_hbm.at[r, sb + 2], srcv.at[b],
                                     semi[b])
                    pltpu.async_copy(dst8_hbm.at[r, sb + 2], dstv.at[b],
                                     semi[b])

                for k in range(NSLOT - 1):
                    fire(nb, k, k)
        return ()

    lax.fori_loop(0, NSUP // 2, super_body, ())

    pltpu.sync_copy(acc, part_hbm.at[r, g])
    plsc.subcore_barrier()

    oct_ = lax.rem(s, 8)
    gm = 2 * c + s // 8
    for rp in range(NR):
        pltpu.sync_copy(part_hbm.at[rp, gm, pl.ds(oct_ * OPF, OPF)],
                        acc.at[pl.ds(rp * OPF, OPF)])

    def red_body(pos, _):
        v = acc[pl.ds(pos * 16, 16)]
        for rp in range(1, NR):
            v = v + acc[pl.ds(rp * OPF + pos * 16, 16)]
        rowv = pos * 2 + half
        plsc.store_scatter(acc2, [rowv, feat], v)
        return ()

    lax.fori_loop(0, OPF // 16, red_body, ())
    pltpu.sync_copy(acc2, out_hbm.at[pl.ds(oct_ * OCT, OCT),
                                     pl.ds(gm * FG, FG)])


@functools.lru_cache(maxsize=None)
def _sc_kernels():
    mesh = plsc.VectorSubcoreMesh(
        core_axis_name="c", subcore_axis_name="s",
        num_cores=NC, num_subcores=NS)
    deg_k = pl.kernel(
        _deg_body,
        out_type=jax.ShapeDtypeStruct((NC, N_PAD, 8), jnp.float32),
        mesh=mesh,
        compiler_params=pltpu.CompilerParams(use_tc_tiling_on_sc=False),
        scratch_types=[
            pltpu.VMEM((NCHUNK_W, CHUNK), jnp.int32),
            pltpu.VMEM((CHUNK, 8), jnp.float32),
            pltpu.VMEM_SHARED((N_PAD, 8), jnp.float32),
        ],
    )
    agg_k = pl.kernel(
        _agg_body,
        out_type=(
            jax.ShapeDtypeStruct((NR, NGRP, NPF), jnp.float32),
            jax.ShapeDtypeStruct((N_PAD, F), jnp.float32),
        ),
        mesh=mesh,
        compiler_params=pltpu.CompilerParams(
            use_tc_tiling_on_sc=False, needs_layout_passes=False),
        scratch_types=[
            pltpu.VMEM((2, SUP, CHUNK), jnp.int32),
            pltpu.VMEM((2, SUP * CHUNK), jnp.int32),
            pltpu.VMEM((CHUNK, FG), jnp.float32),
            pltpu.VMEM((CHUNK, FG), jnp.float32),
            pltpu.VMEM((CHUNK, FG), jnp.float32),
            pltpu.VMEM((CHUNK, FG), jnp.float32),
            pltpu.VMEM((NPF,), jnp.float32),
            pltpu.VMEM((OCT, FG), jnp.float32),
            pltpu.SemaphoreType.DMA,
            pltpu.SemaphoreType.DMA,
            pltpu.SemaphoreType.DMA,
            pltpu.SemaphoreType.DMA,
            pltpu.SemaphoreType.DMA,
            pltpu.SemaphoreType.DMA,
        ],
    )
    return deg_k, agg_k


_BLK = 1264


def _tcmm_body(x_ref, w_ref, h_ref):
    h_ref[...] = jnp.dot(x_ref[...], w_ref[...],
                         preferred_element_type=jnp.float32)


def _tc_matmul1(x_pad, W1):
    return pl.pallas_call(
        _tcmm_body,
        grid=(N_PAD // _BLK,),
        in_specs=[
            pl.BlockSpec((_BLK, D_IN), lambda i: (i, 0)),
            pl.BlockSpec((D_IN, F), lambda i: (0, 0)),
        ],
        out_specs=pl.BlockSpec((_BLK, F), lambda i: (i, 0)),
        out_shape=jax.ShapeDtypeStruct((N_PAD, F), jnp.float32),
    )(x_pad, W1)


def _tc1_body(h_ref, degp_ref, hp_ref, dinv_ref, grp_ref):
    deg = degp_ref[0] + degp_ref[1] + 1.0
    dinv = lax.rsqrt(deg)
    hp = h_ref[...] * dinv
    hp_ref[...] = hp
    dinv_ref[...] = dinv
    grp_ref[...] = jnp.transpose(hp.reshape(_BLK, NGRP, FG), (1, 0, 2))


def _tc_stage1(h1, degp):
    return pl.pallas_call(
        _tc1_body,
        grid=(N_PAD // _BLK,),
        in_specs=[
            pl.BlockSpec((_BLK, F), lambda i: (i, 0)),
            pl.BlockSpec((NC, _BLK, 1), lambda i: (0, i, 0)),
        ],
        out_specs=[
            pl.BlockSpec((_BLK, F), lambda i: (i, 0)),
            pl.BlockSpec((_BLK, 1), lambda i: (i, 0)),
            pl.BlockSpec((NGRP, _BLK, FG), lambda i: (0, i, 0)),
        ],
        out_shape=[
            jax.ShapeDtypeStruct((N_PAD, F), jnp.float32),
            jax.ShapeDtypeStruct((N_PAD, 1), jnp.float32),
            jax.ShapeDtypeStruct((NGRP, N_PAD, FG), jnp.float32),
        ],
    )(h1, degp)


def _tc_combine_body(p_ref, hp_ref, dinv_ref, b_ref, g_ref, be_ref, w_ref,
                     out_ref, grp_ref):
    dinv = dinv_ref[...]
    agg = (p_ref[...] + hp_ref[...]) * dinv + b_ref[...]
    h = jnp.maximum(agg, 0.0)
    mu = jnp.mean(h, axis=1, keepdims=True)
    var = jnp.mean((h - mu) * (h - mu), axis=1, keepdims=True)
    hn = (h - mu) * lax.rsqrt(var + 1e-5) * g_ref[...] + be_ref[...]
    hpn = jnp.dot(hn, w_ref[...], preferred_element_type=jnp.float32) * dinv
    out_ref[...] = hpn
    grp_ref[...] = jnp.transpose(hpn.reshape(_BLK, NGRP, FG), (1, 0, 2))


def _tc_combine(p, hp, dinv2, b, g, be, Wn):
    return pl.pallas_call(
        _tc_combine_body,
        grid=(N_PAD // _BLK,),
        in_specs=[
            pl.BlockSpec((_BLK, F), lambda i: (i, 0)),
            pl.BlockSpec((_BLK, F), lambda i: (i, 0)),
            pl.BlockSpec((_BLK, 1), lambda i: (i, 0)),
            pl.BlockSpec((1, F), lambda i: (0, 0)),
            pl.BlockSpec((1, F), lambda i: (0, 0)),
            pl.BlockSpec((1, F), lambda i: (0, 0)),
            pl.BlockSpec((F, F), lambda i: (0, 0)),
        ],
        out_specs=[
            pl.BlockSpec((_BLK, F), lambda i: (i, 0)),
            pl.BlockSpec((NGRP, _BLK, FG), lambda i: (0, i, 0)),
        ],
        out_shape=[
            jax.ShapeDtypeStruct((N_PAD, F), jnp.float32),
            jax.ShapeDtypeStruct((NGRP, N_PAD, FG), jnp.float32),
        ],
    )(p, hp, dinv2, b, g, be, Wn)


def _tc_final_body(p_ref, hp_ref, dinv_ref, b3_ref, wp1_ref, bp1_ref,
                   wp2_ref, bp2_ref, emb_ref, lsm_ref):
    dinv = dinv_ref[...]
    emb = (p_ref[...] + hp_ref[...]) * dinv + b3_ref[...]
    emb_ref[...] = emb
    r = jnp.maximum(emb, 0.0)
    t = jnp.dot(r, wp1_ref[...], preferred_element_type=jnp.float32)
    t = t + bp1_ref[...]
    u = jnp.dot(t, wp2_ref[...], preferred_element_type=jnp.float32)
    u = u + bp2_ref[...]
    m = jnp.max(u, axis=1, keepdims=True)
    lse = jnp.log(jnp.sum(jnp.exp(u - m), axis=1, keepdims=True)) + m
    lsm_ref[...] = u - lse


_FBLK = 1000


def _tc_final(p, hp, dinv2, b3, Wp1, bp1, Wp2, bp2):
    return pl.pallas_call(
        _tc_final_body,
        grid=(N // _FBLK,),
        in_specs=[
            pl.BlockSpec((_FBLK, F), lambda i: (i, 0)),
            pl.BlockSpec((_FBLK, F), lambda i: (i, 0)),
            pl.BlockSpec((_FBLK, 1), lambda i: (i, 0)),
            pl.BlockSpec((1, F), lambda i: (0, 0)),
            pl.BlockSpec((F, F), lambda i: (0, 0)),
            pl.BlockSpec((1, F), lambda i: (0, 0)),
            pl.BlockSpec((F, OUT), lambda i: (0, 0)),
            pl.BlockSpec((1, OUT), lambda i: (0, 0)),
        ],
        out_specs=[
            pl.BlockSpec((_FBLK, F), lambda i: (i, 0)),
            pl.BlockSpec((_FBLK, OUT), lambda i: (i, 0)),
        ],
        out_shape=[
            jax.ShapeDtypeStruct((N, F), jnp.float32),
            jax.ShapeDtypeStruct((N, OUT), jnp.float32),
        ],
    )(p, hp, dinv2, b3, Wp1, bp1, Wp2, bp2)


def _group(hp):
    return jnp.transpose(hp.reshape(N_PAD, NGRP, FG), (1, 0, 2))


def kernel(x, edge_index, W1, b1, g1, be1, W2, b2, g2, be2, W3, b3,
           Wp1, bp1, Wp2, bp2):
    src = edge_index[0]
    dst = edge_index[1]
    pad_r = EDGES_PER_R - E // NR
    src_r = jnp.pad(src.reshape(NR, E // NR), ((0, 0), (0, pad_r)))
    src_r = src_r.reshape(NR, NSUP, SUP, CHUNK)
    dst8_r = jnp.pad((dst * 8).reshape(NR, E // NR), ((0, 0), (0, pad_r)),
                     constant_values=N * 8)
    dst8_r = dst8_r.reshape(NR, NSUP, SUP * CHUNK)

    dst_w = jnp.pad(dst, (0, E_PAD - E),
                    constant_values=N).reshape(NW, NCHUNK_W, CHUNK)

    zeros8 = jnp.zeros((N_PAD, 8), jnp.float32)
    ones8 = jnp.ones((CHUNK, 8), jnp.float32)
    zerosG = jnp.zeros((NPF,), jnp.float32)
    x_pad = jnp.pad(x, ((0, N_PAD - N), (0, 0)))

    _deg_kernel, _agg_kernel = _sc_kernels()
    degp8 = _deg_kernel(dst_w, zeros8, ones8)
    degp = degp8[:, :, :1]

    hp1, dinv2, hp1g = _tc_stage1(_tc_matmul1(x_pad, W1), degp)

    p1 = _agg_kernel(src_r, dst8_r, hp1g, zerosG)[1]
    hp2, hp2g = _tc_combine(p1, hp1, dinv2, b1.reshape(1, F),
                            g1.reshape(1, F), be1.reshape(1, F), W2)

    p2 = _agg_kernel(src_r, dst8_r, hp2g, zerosG)[1]
    hp3, hp3g = _tc_combine(p2, hp2, dinv2, b2.reshape(1, F),
                            g2.reshape(1, F), be2.reshape(1, F), W3)

    p3 = _agg_kernel(src_r, dst8_r, hp3g, zerosG)[1]
    emb, lsm = _tc_final(p3, hp3, dinv2, b3.reshape(1, F), Wp1,
                         bp1.reshape(1, F), Wp2, bp2.reshape(1, OUT))
    return (emb, lsm)


# speedup vs baseline: 1.0399x; 1.0072x over previous
import functools

import jax
import jax.numpy as jnp
from jax import lax
from jax.experimental import pallas as pl
from jax.experimental.pallas import tpu as pltpu
from jax.experimental.pallas import tpu_sc as plsc

N = 10000
E = 320000
D_IN = 128
F = 32
OUT = 128

NC = 2
NS = 16
NW = NC * NS
CHUNK = 128
N_PAD = 10112
ROWS_PER_TILE = N_PAD // NS

FG = 8
NGRP = F // FG
NR = NW // NGRP
SUP = 16
EDGES_PER_R = 40960
NCHUNK_R = EDGES_PER_R // CHUNK
NSUP = NCHUNK_R // SUP
E_PAD = EDGES_PER_R * NR

EDGES_PER_W = E_PAD // NW
NCHUNK_W = EDGES_PER_W // CHUNK


def _deg_body(dst_hbm, zeros_hbm, ones_hbm, out_hbm, dst_v, ones_v, deg_sh):
    c = lax.axis_index("c")
    s = lax.axis_index("s")
    wid = s * NC + c
    row0 = s * ROWS_PER_TILE
    pltpu.sync_copy(zeros_hbm.at[pl.ds(row0, ROWS_PER_TILE)],
                    deg_sh.at[pl.ds(row0, ROWS_PER_TILE)])
    pltpu.sync_copy(dst_hbm.at[wid], dst_v)
    pltpu.sync_copy(ones_hbm, ones_v)
    plsc.subcore_barrier()

    def body(j, _):
        pltpu.sync_copy(ones_v, deg_sh.at[dst_v.at[j]], add=True)
        return ()

    lax.fori_loop(0, NCHUNK_W, body, ())
    plsc.subcore_barrier()
    pltpu.sync_copy(deg_sh.at[pl.ds(row0, ROWS_PER_TILE)],
                    out_hbm.at[c, pl.ds(row0, ROWS_PER_TILE)])


NSLOT = 4
NPF = N_PAD * FG
OCT = N_PAD // 8
OPF = OCT * FG


def _agg_body(src_hbm, dst8_hbm, hp_hbm, zeros_hbm, part_hbm, out_hbm,
              srcv, dstv, rows0, rows1, rows2, rows3, acc, acc2,
              semi0, semi1, semg0, semg1, semg2, semg3):
    c = lax.axis_index("c")
    s = lax.axis_index("s")
    g = 2 * c + s // 8
    r = lax.rem(s, 8)
    semi = [semi0, semi1]
    semg = [semg0, semg1, semg2, semg3]
    rows = [rows0, rows1, rows2, rows3]

    pltpu.sync_copy(zeros_hbm, acc)

    iota = lax.iota(jnp.int32, 16)
    feat = jnp.bitwise_and(iota, 7)
    half = lax.shift_right_logical(iota, 3)
    pats = [2 * j + half for j in range(8)]
    evecs = [[t * 16 + 2 * j + half for j in range(8)] for t in range(8)]

    def fire(b, ci, k):
        pltpu.async_copy(hp_hbm.at[g].at[srcv.at[b, ci]], rows[k], semg[k])

    def wait(b, ci, k):
        pltpu.make_async_copy(
            hp_hbm.at[g].at[srcv.at[b, ci]], rows[k], semg[k]).wait()

    pltpu.async_copy(src_hbm.at[r, 0], srcv.at[0], semi0)
    pltpu.async_copy(dst8_hbm.at[r, 0], dstv.at[0], semi0)
    pltpu.make_async_copy(src_hbm.at[r, 0], srcv.at[0], semi0).wait()
    pltpu.make_async_copy(dst8_hbm.at[r, 0], dstv.at[0], semi0).wait()
    pltpu.async_copy(src_hbm.at[r, 1], srcv.at[1], semi1)
    pltpu.async_copy(dst8_hbm.at[r, 1], dstv.at[1], semi1)
    for k in range(NSLOT - 1):
        fire(0, k, k)

    def scatter_chunk(b, ci, k):
        for t in range(8):
            d16 = dstv[b, pl.ds(ci * CHUNK + t * 16, 16)]
            addrs = [jnp.bitwise_or(
                d16.at[pats[j]].get(mode="promise_in_bounds"), feat)
                for j in range(8)]
            vs = [plsc.load_gather(rows[k], [evecs[t][j], feat])
                  for j in range(8)]
            for j in range(8):
                plsc.addupdate_scatter(acc, [addrs[j]], vs[j])

    def super_body(sj, _):
        for b in range(2):
            sb = sj * 2 + b

            def chunk_body(cj, _):
                for k in range(NSLOT):
                    ci = cj * NSLOT + k
                    wait(b, ci, k)

                    @pl.when(ci + NSLOT - 1 < SUP)
                    def _():
                        fire(b, ci + NSLOT - 1, (k + NSLOT - 1) % NSLOT)

                    scatter_chunk(b, ci, k)
                return ()

            lax.fori_loop(0, SUP // NSLOT, chunk_body, ())

            @pl.when(sb + 1 < NSUP)
            def _():
                nb = 1 - b
                pltpu.make_async_copy(
                    src_hbm.at[r, sb + 1], srcv.at[nb], semi[nb]).wait()
                pltpu.make_async_copy(
                    dst8_hbm.at[r, sb + 1], dstv.at[nb], semi[nb]).wait()

                @pl.when(sb + 2 < NSUP)
                def _():
                    pltpu.async_copy(src_hbm.at[r, sb + 2], srcv.at[b],
                                     semi[b])
                    pltpu.async_copy(dst8_hbm.at[r, sb + 2], dstv.at[b],
                                     semi[b])

                for k in range(NSLOT - 1):
                    fire(nb, k, k)
        return ()

    lax.fori_loop(0, NSUP // 2, super_body, ())

    pltpu.sync_copy(acc, part_hbm.at[r, g])
    plsc.subcore_barrier()

    oct_ = lax.rem(s, 8)
    gm = 2 * c + s // 8
    for rp in range(NR):
        pltpu.sync_copy(part_hbm.at[rp, gm, pl.ds(oct_ * OPF, OPF)],
                        acc.at[pl.ds(rp * OPF, OPF)])

    def red_body(pos, _):
        v = acc[pl.ds(pos * 16, 16)]
        for rp in range(1, NR):
            v = v + acc[pl.ds(rp * OPF + pos * 16, 16)]
        rowv = pos * 2 + half
        plsc.store_scatter(acc2, [rowv, feat], v)
        return ()

    lax.fori_loop(0, OPF // 16, red_body, ())
    pltpu.sync_copy(acc2, out_hbm.at[pl.ds(oct_ * OCT, OCT),
                                     pl.ds(gm * FG, FG)])


@functools.lru_cache(maxsize=None)
def _sc_kernels():
    mesh = plsc.VectorSubcoreMesh(
        core_axis_name="c", subcore_axis_name="s",
        num_cores=NC, num_subcores=NS)
    deg_k = pl.kernel(
        _deg_body,
        out_type=jax.ShapeDtypeStruct((NC, N_PAD, 8), jnp.float32),
        mesh=mesh,
        compiler_params=pltpu.CompilerParams(use_tc_tiling_on_sc=False),
        scratch_types=[
            pltpu.VMEM((NCHUNK_W, CHUNK), jnp.int32),
            pltpu.VMEM((CHUNK, 8), jnp.float32),
            pltpu.VMEM_SHARED((N_PAD, 8), jnp.float32),
        ],
    )
    agg_k = pl.kernel(
        _agg_body,
        out_type=(
            jax.ShapeDtypeStruct((NR, NGRP, NPF), jnp.float32),
            jax.ShapeDtypeStruct((N_PAD, F), jnp.float32),
        ),
        mesh=mesh,
        compiler_params=pltpu.CompilerParams(
            use_tc_tiling_on_sc=False, needs_layout_passes=False),
        scratch_types=[
            pltpu.VMEM((2, SUP, CHUNK), jnp.int32),
            pltpu.VMEM((2, SUP * CHUNK), jnp.int32),
            pltpu.VMEM((CHUNK, FG), jnp.float32),
            pltpu.VMEM((CHUNK, FG), jnp.float32),
            pltpu.VMEM((CHUNK, FG), jnp.float32),
            pltpu.VMEM((CHUNK, FG), jnp.float32),
            pltpu.VMEM((NPF,), jnp.float32),
            pltpu.VMEM((OCT, FG), jnp.float32),
            pltpu.SemaphoreType.DMA,
            pltpu.SemaphoreType.DMA,
            pltpu.SemaphoreType.DMA,
            pltpu.SemaphoreType.DMA,
            pltpu.SemaphoreType.DMA,
            pltpu.SemaphoreType.DMA,
        ],
    )
    return deg_k, agg_k


_BLK = 1264


def _tcmm_body(x_ref, w_ref, h_ref):
    h_ref[...] = jnp.dot(x_ref[...], w_ref[...],
                         preferred_element_type=jnp.float32)


def _tc_matmul1(x_pad, W1):
    return pl.pallas_call(
        _tcmm_body,
        grid=(N_PAD // _BLK,),
        in_specs=[
            pl.BlockSpec((_BLK, D_IN), lambda i: (i, 0)),
            pl.BlockSpec((D_IN, F), lambda i: (0, 0)),
        ],
        out_specs=pl.BlockSpec((_BLK, F), lambda i: (i, 0)),
        out_shape=jax.ShapeDtypeStruct((N_PAD, F), jnp.float32),
    )(x_pad, W1)


def _tc1_body(h_ref, degp_ref, hp_ref, dinv_ref, grp_ref):
    deg = degp_ref[0] + degp_ref[1] + 1.0
    dinv = lax.rsqrt(deg)
    hp = h_ref[...] * dinv
    hp_ref[...] = hp
    dinv_ref[...] = dinv
    for gg in range(NGRP):
        grp_ref[gg] = hp[:, gg * FG:(gg + 1) * FG]


def _tc_stage1(h1, degp):
    return pl.pallas_call(
        _tc1_body,
        grid=(N_PAD // _BLK,),
        in_specs=[
            pl.BlockSpec((_BLK, F), lambda i: (i, 0)),
            pl.BlockSpec((NC, _BLK, 1), lambda i: (0, i, 0)),
        ],
        out_specs=[
            pl.BlockSpec((_BLK, F), lambda i: (i, 0)),
            pl.BlockSpec((_BLK, 1), lambda i: (i, 0)),
            pl.BlockSpec((NGRP, _BLK, FG), lambda i: (0, i, 0)),
        ],
        out_shape=[
            jax.ShapeDtypeStruct((N_PAD, F), jnp.float32),
            jax.ShapeDtypeStruct((N_PAD, 1), jnp.float32),
            jax.ShapeDtypeStruct((NGRP, N_PAD, FG), jnp.float32),
        ],
    )(h1, degp)


def _tc_combine_body(p_ref, hp_ref, dinv_ref, b_ref, g_ref, be_ref, w_ref,
                     out_ref, grp_ref):
    dinv = dinv_ref[...]
    agg = (p_ref[...] + hp_ref[...]) * dinv + b_ref[...]
    h = jnp.maximum(agg, 0.0)
    mu = jnp.mean(h, axis=1, keepdims=True)
    var = jnp.mean((h - mu) * (h - mu), axis=1, keepdims=True)
    hn = (h - mu) * lax.rsqrt(var + 1e-5) * g_ref[...] + be_ref[...]
    hpn = jnp.dot(hn, w_ref[...], preferred_element_type=jnp.float32) * dinv
    out_ref[...] = hpn
    for gg in range(NGRP):
        grp_ref[gg] = hpn[:, gg * FG:(gg + 1) * FG]


def _tc_combine(p, hp, dinv2, b, g, be, Wn):
    return pl.pallas_call(
        _tc_combine_body,
        grid=(N_PAD // _BLK,),
        in_specs=[
            pl.BlockSpec((_BLK, F), lambda i: (i, 0)),
            pl.BlockSpec((_BLK, F), lambda i: (i, 0)),
            pl.BlockSpec((_BLK, 1), lambda i: (i, 0)),
            pl.BlockSpec((1, F), lambda i: (0, 0)),
            pl.BlockSpec((1, F), lambda i: (0, 0)),
            pl.BlockSpec((1, F), lambda i: (0, 0)),
            pl.BlockSpec((F, F), lambda i: (0, 0)),
        ],
        out_specs=[
            pl.BlockSpec((_BLK, F), lambda i: (i, 0)),
            pl.BlockSpec((NGRP, _BLK, FG), lambda i: (0, i, 0)),
        ],
        out_shape=[
            jax.ShapeDtypeStruct((N_PAD, F), jnp.float32),
            jax.ShapeDtypeStruct((NGRP, N_PAD, FG), jnp.float32),
        ],
    )(p, hp, dinv2, b, g, be, Wn)


def _tc_final_body(p_ref, hp_ref, dinv_ref, b3_ref, wp1_ref, bp1_ref,
                   wp2_ref, bp2_ref, emb_ref, lsm_ref):
    dinv = dinv_ref[...]
    emb = (p_ref[...] + hp_ref[...]) * dinv + b3_ref[...]
    emb_ref[...] = emb
    r = jnp.maximum(emb, 0.0)
    t = jnp.dot(r, wp1_ref[...], preferred_element_type=jnp.float32)
    t = t + bp1_ref[...]
    u = jnp.dot(t, wp2_ref[...], preferred_element_type=jnp.float32)
    u = u + bp2_ref[...]
    m = jnp.max(u, axis=1, keepdims=True)
    lse = jnp.log(jnp.sum(jnp.exp(u - m), axis=1, keepdims=True)) + m
    lsm_ref[...] = u - lse


_FBLK = 1000


def _tc_final(p, hp, dinv2, b3, Wp1, bp1, Wp2, bp2):
    return pl.pallas_call(
        _tc_final_body,
        grid=(N // _FBLK,),
        in_specs=[
            pl.BlockSpec((_FBLK, F), lambda i: (i, 0)),
            pl.BlockSpec((_FBLK, F), lambda i: (i, 0)),
            pl.BlockSpec((_FBLK, 1), lambda i: (i, 0)),
            pl.BlockSpec((1, F), lambda i: (0, 0)),
            pl.BlockSpec((F, F), lambda i: (0, 0)),
            pl.BlockSpec((1, F), lambda i: (0, 0)),
            pl.BlockSpec((F, OUT), lambda i: (0, 0)),
            pl.BlockSpec((1, OUT), lambda i: (0, 0)),
        ],
        out_specs=[
            pl.BlockSpec((_FBLK, F), lambda i: (i, 0)),
            pl.BlockSpec((_FBLK, OUT), lambda i: (i, 0)),
        ],
        out_shape=[
            jax.ShapeDtypeStruct((N, F), jnp.float32),
            jax.ShapeDtypeStruct((N, OUT), jnp.float32),
        ],
    )(p, hp, dinv2, b3, Wp1, bp1, Wp2, bp2)


def _group(hp):
    return jnp.transpose(hp.reshape(N_PAD, NGRP, FG), (1, 0, 2))


def kernel(x, edge_index, W1, b1, g1, be1, W2, b2, g2, be2, W3, b3,
           Wp1, bp1, Wp2, bp2):
    src = edge_index[0]
    dst = edge_index[1]
    pad_r = EDGES_PER_R - E // NR
    src_r = jnp.pad(src.reshape(NR, E // NR), ((0, 0), (0, pad_r)))
    src_r = src_r.reshape(NR, NSUP, SUP, CHUNK)
    dst8_r = jnp.pad((dst * 8).reshape(NR, E // NR), ((0, 0), (0, pad_r)),
                     constant_values=N * 8)
    dst8_r = dst8_r.reshape(NR, NSUP, SUP * CHUNK)

    dst_w = jnp.pad(dst, (0, E_PAD - E),
                    constant_values=N).reshape(NW, NCHUNK_W, CHUNK)

    zeros8 = jnp.zeros((N_PAD, 8), jnp.float32)
    ones8 = jnp.ones((CHUNK, 8), jnp.float32)
    zerosG = jnp.zeros((NPF,), jnp.float32)
    x_pad = jnp.pad(x, ((0, N_PAD - N), (0, 0)))

    _deg_kernel, _agg_kernel = _sc_kernels()
    degp8 = _deg_kernel(dst_w, zeros8, ones8)
    degp = degp8[:, :, :1]

    hp1, dinv2, hp1g = _tc_stage1(_tc_matmul1(x_pad, W1), degp)

    p1 = _agg_kernel(src_r, dst8_r, hp1g, zerosG)[1]
    hp2, hp2g = _tc_combine(p1, hp1, dinv2, b1.reshape(1, F),
                            g1.reshape(1, F), be1.reshape(1, F), W2)

    p2 = _agg_kernel(src_r, dst8_r, hp2g, zerosG)[1]
    hp3, hp3g = _tc_combine(p2, hp2, dinv2, b2.reshape(1, F),
                            g2.reshape(1, F), be2.reshape(1, F), W3)

    p3 = _agg_kernel(src_r, dst8_r, hp3g, zerosG)[1]
    emb, lsm = _tc_final(p3, hp3, dinv2, b3.reshape(1, F), Wp1,
                         bp1.reshape(1, F), Wp2, bp2.reshape(1, OUT))
    return (emb, lsm)
